# baseline (speedup 1.0000x reference)
import functools

import jax
import jax.numpy as jnp
from jax import lax
from jax.experimental import pallas as pl
from jax.experimental.pallas import tpu as pltpu
from jax.experimental.pallas import tpu_sc as plsc


def _encoder_body(x_ref, w1_ref, b1_ref, w2_ref, b2_ref, w3_ref, b3_ref,
                  qi_ref):
    x = x_ref[...]
    h = jnp.maximum(
        jnp.dot(x, w1_ref[...], preferred_element_type=jnp.float32)
        + b1_ref[...], 0.0)
    h = jnp.maximum(
        jnp.dot(h, w2_ref[...], preferred_element_type=jnp.float32)
        + b2_ref[...], 0.0)
    qi_ref[...] = (
        jnp.dot(h, w3_ref[...], preferred_element_type=jnp.float32)
        + b3_ref[...])


def _encoder_call(x, w1, b1, w2, b2, w3, b3, tb=512):
    bsz, d = x.shape
    h = w1.shape[1]
    l = w3.shape[1]
    grid = (bsz // tb,)
    return pl.pallas_call(
        _encoder_body,
        grid=grid,
        in_specs=[
            pl.BlockSpec((tb, d), lambda i: (i, 0)),
            pl.BlockSpec((d, h), lambda i: (0, 0)),
            pl.BlockSpec((1, h), lambda i: (0, 0)),
            pl.BlockSpec((h, h), lambda i: (0, 0)),
            pl.BlockSpec((1, h), lambda i: (0, 0)),
            pl.BlockSpec((h, l), lambda i: (0, 0)),
            pl.BlockSpec((1, l), lambda i: (0, 0)),
        ],
        out_specs=pl.BlockSpec((tb, l), lambda i: (i, 0)),
        out_shape=jax.ShapeDtypeStruct((bsz, l), jnp.float32),
    )(x, w1, b1, w2, b2, w3, b3)


def _argmin_body(qi_ref, emb_ref, idx_ref, dmin_ref, *, kc):
    q = qi_ref[...]
    tb = q.shape[0]
    k = emb_ref.shape[0]
    qsq = jnp.sum(q * q, axis=1, keepdims=True)
    best = jnp.full((tb, 1), jnp.inf, dtype=jnp.float32)
    besti = jnp.zeros((tb, 1), dtype=jnp.int32)
    for c in range(k // kc):
        e = emb_ref[c * kc:(c + 1) * kc, :]
        esq = jnp.sum(e * e, axis=1)
        mm = lax.dot_general(
            q, e, dimension_numbers=(((1,), (1,)), ((), ())),
            preferred_element_type=jnp.float32)
        d2 = (qsq - 2.0 * mm) + esq[None, :]
        dist = jnp.sqrt(jnp.maximum(d2, 0.0))
        cmin = jnp.min(dist, axis=1, keepdims=True)
        iot = lax.broadcasted_iota(jnp.int32, (tb, kc), 1)
        cidx = jnp.min(jnp.where(dist == cmin, iot, jnp.int32(k)),
                       axis=1, keepdims=True) + c * kc
        upd = cmin < best
        besti = jnp.where(upd, cidx, besti)
        best = jnp.where(upd, cmin, best)
    idx_ref[...] = besti
    dmin_ref[...] = best


def _argmin_call(qi, emb, tb=512, kc=1024):
    bsz, l = qi.shape
    k = emb.shape[0]
    grid = (bsz // tb,)
    return pl.pallas_call(
        functools.partial(_argmin_body, kc=kc),
        grid=grid,
        in_specs=[
            pl.BlockSpec((tb, l), lambda i: (i, 0)),
            pl.BlockSpec((k, l), lambda i: (0, 0)),
        ],
        out_specs=[
            pl.BlockSpec((tb, 1), lambda i: (i, 0)),
            pl.BlockSpec((tb, 1), lambda i: (i, 0)),
        ],
        out_shape=[
            jax.ShapeDtypeStruct((bsz, 1), jnp.int32),
            jax.ShapeDtypeStruct((bsz, 1), jnp.float32),
        ],
    )(qi, emb)


def _onehot_body(idx_ref, oh_ref, cnt_ref):
    bb = pl.program_id(1)
    kb = pl.program_id(0)
    tbo, kco = oh_ref.shape
    idxcol = idx_ref[...]
    iot = lax.broadcasted_iota(jnp.int32, (tbo, kco), 1) + kb * kco
    oh = jnp.where(iot == idxcol, 1.0, 0.0).astype(jnp.float32)
    oh_ref[...] = oh

    @pl.when(bb == 0)
    def _():
        cnt_ref[...] = jnp.zeros_like(cnt_ref)

    cnt_ref[...] += jnp.sum(oh, axis=0, keepdims=True)


def _onehot_call(idx, k, tbo=512, kco=2048):
    bsz = idx.shape[0]
    grid = (k // kco, bsz // tbo)
    return pl.pallas_call(
        _onehot_body,
        grid=grid,
        in_specs=[pl.BlockSpec((tbo, 1), lambda kb, bb: (bb, 0))],
        out_specs=[
            pl.BlockSpec((tbo, kco), lambda kb, bb: (bb, kb)),
            pl.BlockSpec((1, kco), lambda kb, bb: (0, kb)),
        ],
        out_shape=[
            jax.ShapeDtypeStruct((bsz, k), jnp.float32),
            jax.ShapeDtypeStruct((1, k), jnp.float32),
        ],
    )(idx)


def _dectable_body(emb_ref, w4_ref, b4_ref, w5_ref, b5_ref, dec_ref):
    t = jnp.maximum(
        jnp.dot(emb_ref[...], w4_ref[...],
                preferred_element_type=jnp.float32) + b4_ref[...], 0.0)
    dec_ref[...] = (
        jnp.dot(t, w5_ref[...], preferred_element_type=jnp.float32)
        + b5_ref[...])


def _dectable_call(emb, w4, b4, w5, b5, tk=2048):
    k, l = emb.shape
    h = w4.shape[1]
    d = w5.shape[1]
    grid = (k // tk,)
    return pl.pallas_call(
        _dectable_body,
        grid=grid,
        in_specs=[
            pl.BlockSpec((tk, l), lambda i: (i, 0)),
            pl.BlockSpec((l, h), lambda i: (0, 0)),
            pl.BlockSpec((1, h), lambda i: (0, 0)),
            pl.BlockSpec((h, d), lambda i: (0, 0)),
            pl.BlockSpec((1, d), lambda i: (0, 0)),
        ],
        out_specs=pl.BlockSpec((tk, d), lambda i: (i, 0)),
        out_shape=jax.ShapeDtypeStruct((k, d), jnp.float32),
    )(emb, w4, b4, w5, b5)


def _sc_gather_call(idx3, dec):
    nw, nj, jc = idx3.shape
    bpw = nj * jc
    bsz = nw * bpw
    d = dec.shape[1]
    mesh = plsc.VectorSubcoreMesh(core_axis_name="c", subcore_axis_name="s")

    @functools.partial(
        pl.kernel,
        mesh=mesh,
        out_type=jax.ShapeDtypeStruct((bsz, d), jnp.float32),
        scratch_types=[
            pltpu.VMEM((nj, jc), jnp.int32),
            pltpu.VMEM((bpw, d), jnp.float32),
            pltpu.SemaphoreType.DMA,
        ],
    )
    def sc_kernel(idx_hbm, dec_hbm, out_hbm, idx_v, dec_v, sem):
        wid = lax.axis_index("s") * 2 + lax.axis_index("c")
        base = wid * bpw
        pltpu.sync_copy(idx_hbm.at[wid], idx_v)
        copies = []
        for j in range(nj):
            copies.append(pltpu.async_copy(
                dec_hbm.at[idx_v.at[j]], dec_v.at[pl.ds(j * jc, jc)], sem))
        for cp in copies:
            cp.wait()
        pltpu.sync_copy(dec_v, out_hbm.at[pl.ds(base, bpw)])

    return sc_kernel(idx3, dec)


def _finalize_body(dmin_ref, cnt_ref, ql_ref, perp_ref, *, beta, nelem):
    dmin = dmin_ref[...]
    cb = jnp.sum(dmin * dmin) * (1.0 / nelem)
    ql_ref[...] = jnp.reshape(cb + beta * cb, (1, 1))
    bsz = dmin.shape[0]
    e_mean = cnt_ref[...] * (1.0 / bsz)
    plogp = e_mean * jnp.log(e_mean + 1e-10)
    perp_ref[...] = jnp.reshape(jnp.exp(-jnp.sum(plogp)), (1, 1))


def _finalize_call(dmin, counts, l, beta=0.25):
    bsz = dmin.shape[0]
    k = counts.shape[1]
    return pl.pallas_call(
        functools.partial(_finalize_body, beta=beta, nelem=bsz * l),
        grid=(1,),
        in_specs=[
            pl.BlockSpec((bsz, 1), lambda i: (0, 0)),
            pl.BlockSpec((1, k), lambda i: (0, 0)),
        ],
        out_specs=[
            pl.BlockSpec((1, 1), lambda i: (0, 0)),
            pl.BlockSpec((1, 1), lambda i: (0, 0)),
        ],
        out_shape=[
            jax.ShapeDtypeStruct((1, 1), jnp.float32),
            jax.ShapeDtypeStruct((1, 1), jnp.float32),
        ],
    )(dmin, counts)


def kernel(x, W1, b1, W2, b2, W3, b3, emb, W4, b4, W5, b5):
    bsz, d = x.shape
    k, l = emb.shape
    b1r, b2r, b3r = b1.reshape(1, -1), b2.reshape(1, -1), b3.reshape(1, -1)
    b4r, b5r = b4.reshape(1, -1), b5.reshape(1, -1)

    qi = _encoder_call(x, W1, b1r, W2, b2r, W3, b3r)
    idx2, dmin = _argmin_call(qi, emb)
    minenc, counts = _onehot_call(idx2, k)
    dec = _dectable_call(emb, W4, b4r, W5, b5r)
    idx3 = idx2.reshape(bsz // 512, 4, 128)
    out = _sc_gather_call(idx3, dec)
    ql, perp = _finalize_call(dmin, counts, l)
    return (out, ql.reshape(()), perp.reshape(()), minenc, idx2)


# speedup vs baseline: 1.0202x; 1.0202x over previous
import functools

import jax
import jax.numpy as jnp
from jax import lax
from jax.experimental import pallas as pl
from jax.experimental.pallas import tpu as pltpu
from jax.experimental.pallas import tpu_sc as plsc


def _encoder_body(x_ref, w1_ref, b1_ref, w2_ref, b2_ref, w3_ref, b3_ref,
                  qi_ref):
    x = x_ref[...]
    h = jnp.maximum(
        jnp.dot(x, w1_ref[...], preferred_element_type=jnp.float32)
        + b1_ref[...], 0.0)
    h = jnp.maximum(
        jnp.dot(h, w2_ref[...], preferred_element_type=jnp.float32)
        + b2_ref[...], 0.0)
    qi_ref[...] = (
        jnp.dot(h, w3_ref[...], preferred_element_type=jnp.float32)
        + b3_ref[...])


def _encoder_call(x, w1, b1, w2, b2, w3, b3, tb=512):
    bsz, d = x.shape
    h = w1.shape[1]
    l = w3.shape[1]
    grid = (bsz // tb,)
    return pl.pallas_call(
        _encoder_body,
        grid=grid,
        in_specs=[
            pl.BlockSpec((tb, d), lambda i: (i, 0)),
            pl.BlockSpec((d, h), lambda i: (0, 0)),
            pl.BlockSpec((1, h), lambda i: (0, 0)),
            pl.BlockSpec((h, h), lambda i: (0, 0)),
            pl.BlockSpec((1, h), lambda i: (0, 0)),
            pl.BlockSpec((h, l), lambda i: (0, 0)),
            pl.BlockSpec((1, l), lambda i: (0, 0)),
        ],
        out_specs=pl.BlockSpec((tb, l), lambda i: (i, 0)),
        out_shape=jax.ShapeDtypeStruct((bsz, l), jnp.float32),
    )(x, w1, b1, w2, b2, w3, b3)


def _argmin_body(qi_ref, emb_ref, idx_ref, dmin_ref, *, kc):
    q = qi_ref[...]
    tb = q.shape[0]
    k = emb_ref.shape[0]
    qsq = jnp.sum(q * q, axis=1, keepdims=True)
    best = jnp.full((tb, 1), jnp.inf, dtype=jnp.float32)
    besti = jnp.zeros((tb, 1), dtype=jnp.int32)
    for c in range(k // kc):
        e = emb_ref[c * kc:(c + 1) * kc, :]
        esq = jnp.sum(e * e, axis=1)
        mm = lax.dot_general(
            q, e, dimension_numbers=(((1,), (1,)), ((), ())),
            preferred_element_type=jnp.float32)
        d2 = (qsq - 2.0 * mm) + esq[None, :]
        dist = jnp.sqrt(jnp.maximum(d2, 0.0))
        cmin = jnp.min(dist, axis=1, keepdims=True)
        iot = lax.broadcasted_iota(jnp.int32, (tb, kc), 1)
        cidx = jnp.min(jnp.where(dist == cmin, iot, jnp.int32(k)),
                       axis=1, keepdims=True) + c * kc
        upd = cmin < best
        besti = jnp.where(upd, cidx, besti)
        best = jnp.where(upd, cmin, best)
    idx_ref[...] = besti
    dmin_ref[...] = best


def _argmin_call(qi, emb, tb=512, kc=1024):
    bsz, l = qi.shape
    k = emb.shape[0]
    grid = (bsz // tb,)
    return pl.pallas_call(
        functools.partial(_argmin_body, kc=kc),
        grid=grid,
        in_specs=[
            pl.BlockSpec((tb, l), lambda i: (i, 0)),
            pl.BlockSpec((k, l), lambda i: (0, 0)),
        ],
        out_specs=[
            pl.BlockSpec((tb, 1), lambda i: (i, 0)),
            pl.BlockSpec((tb, 1), lambda i: (i, 0)),
        ],
        out_shape=[
            jax.ShapeDtypeStruct((bsz, 1), jnp.int32),
            jax.ShapeDtypeStruct((bsz, 1), jnp.float32),
        ],
    )(qi, emb)


def _onehot_body(idx_ref, oh_ref, cnt_ref):
    bb = pl.program_id(1)
    kb = pl.program_id(0)
    tbo, kco = oh_ref.shape
    idxcol = idx_ref[...]
    iot = lax.broadcasted_iota(jnp.int32, (tbo, kco), 1) + kb * kco
    oh = jnp.where(iot == idxcol, 1.0, 0.0).astype(jnp.float32)
    oh_ref[...] = oh

    @pl.when(bb == 0)
    def _():
        cnt_ref[...] = jnp.zeros_like(cnt_ref)

    cnt_ref[...] += jnp.sum(oh, axis=0, keepdims=True)


def _onehot_call(idx, k, tbo=512, kco=4096):
    bsz = idx.shape[0]
    grid = (k // kco, bsz // tbo)
    return pl.pallas_call(
        _onehot_body,
        grid=grid,
        in_specs=[pl.BlockSpec((tbo, 1), lambda kb, bb: (bb, 0))],
        out_specs=[
            pl.BlockSpec((tbo, kco), lambda kb, bb: (bb, kb)),
            pl.BlockSpec((1, kco), lambda kb, bb: (0, kb)),
        ],
        out_shape=[
            jax.ShapeDtypeStruct((bsz, k), jnp.float32),
            jax.ShapeDtypeStruct((1, k), jnp.float32),
        ],
    )(idx)


def _dectable_body(emb_ref, w4_ref, b4_ref, w5_ref, b5_ref, dec_ref):
    t = jnp.maximum(
        jnp.dot(emb_ref[...], w4_ref[...],
                preferred_element_type=jnp.float32) + b4_ref[...], 0.0)
    dec_ref[...] = (
        jnp.dot(t, w5_ref[...], preferred_element_type=jnp.float32)
        + b5_ref[...])


def _dectable_call(emb, w4, b4, w5, b5, tk=2048):
    k, l = emb.shape
    h = w4.shape[1]
    d = w5.shape[1]
    grid = (k // tk,)
    return pl.pallas_call(
        _dectable_body,
        grid=grid,
        in_specs=[
            pl.BlockSpec((tk, l), lambda i: (i, 0)),
            pl.BlockSpec((l, h), lambda i: (0, 0)),
            pl.BlockSpec((1, h), lambda i: (0, 0)),
            pl.BlockSpec((h, d), lambda i: (0, 0)),
            pl.BlockSpec((1, d), lambda i: (0, 0)),
        ],
        out_specs=pl.BlockSpec((tk, d), lambda i: (i, 0)),
        out_shape=jax.ShapeDtypeStruct((k, d), jnp.float32),
    )(emb, w4, b4, w5, b5)


def _sc_gather_call(idx3, dec):
    nw, nj, jc = idx3.shape
    bpw = nj * jc
    bsz = nw * bpw
    d = dec.shape[1]
    mesh = plsc.VectorSubcoreMesh(core_axis_name="c", subcore_axis_name="s")

    @functools.partial(
        pl.kernel,
        mesh=mesh,
        out_type=jax.ShapeDtypeStruct((bsz, d), jnp.float32),
        scratch_types=[
            pltpu.VMEM((nj, jc), jnp.int32),
            pltpu.VMEM((bpw, d), jnp.float32),
            pltpu.SemaphoreType.DMA,
        ],
    )
    def sc_kernel(idx_hbm, dec_hbm, out_hbm, idx_v, dec_v, sem):
        wid = lax.axis_index("s") * 2 + lax.axis_index("c")
        base = wid * bpw
        pltpu.sync_copy(idx_hbm.at[wid], idx_v)
        copies = []
        for j in range(nj):
            copies.append(pltpu.async_copy(
                dec_hbm.at[idx_v.at[j]], dec_v.at[pl.ds(j * jc, jc)], sem))
        for cp in copies:
            cp.wait()
        pltpu.sync_copy(dec_v, out_hbm.at[pl.ds(base, bpw)])

    return sc_kernel(idx3, dec)


def _finalize_body(dmin_ref, cnt_ref, ql_ref, perp_ref, *, beta, nelem):
    dmin = dmin_ref[...]
    cb = jnp.sum(dmin * dmin) * (1.0 / nelem)
    ql_ref[...] = jnp.reshape(cb + beta * cb, (1, 1))
    bsz = dmin.shape[0]
    e_mean = cnt_ref[...] * (1.0 / bsz)
    plogp = e_mean * jnp.log(e_mean + 1e-10)
    perp_ref[...] = jnp.reshape(jnp.exp(-jnp.sum(plogp)), (1, 1))


def _finalize_call(dmin, counts, l, beta=0.25):
    bsz = dmin.shape[0]
    k = counts.shape[1]
    return pl.pallas_call(
        functools.partial(_finalize_body, beta=beta, nelem=bsz * l),
        grid=(1,),
        in_specs=[
            pl.BlockSpec((bsz, 1), lambda i: (0, 0)),
            pl.BlockSpec((1, k), lambda i: (0, 0)),
        ],
        out_specs=[
            pl.BlockSpec((1, 1), lambda i: (0, 0)),
            pl.BlockSpec((1, 1), lambda i: (0, 0)),
        ],
        out_shape=[
            jax.ShapeDtypeStruct((1, 1), jnp.float32),
            jax.ShapeDtypeStruct((1, 1), jnp.float32),
        ],
    )(dmin, counts)


def kernel(x, W1, b1, W2, b2, W3, b3, emb, W4, b4, W5, b5):
    bsz, d = x.shape
    k, l = emb.shape
    b1r, b2r, b3r = b1.reshape(1, -1), b2.reshape(1, -1), b3.reshape(1, -1)
    b4r, b5r = b4.reshape(1, -1), b5.reshape(1, -1)

    dec = _dectable_call(emb, W4, b4r, W5, b5r)
    qi = _encoder_call(x, W1, b1r, W2, b2r, W3, b3r)
    idx2, dmin = _argmin_call(qi, emb)
    idx3 = idx2.reshape(bsz // 512, 4, 128)
    out = _sc_gather_call(idx3, dec)
    minenc, counts = _onehot_call(idx2, k)
    ql, perp = _finalize_call(dmin, counts, l)
    return (out, ql.reshape(()), perp.reshape(()), minenc, idx2)
